# baseline (speedup 1.0000x reference)
import functools

import jax
import jax.numpy as jnp
from jax import lax
from jax.experimental import pallas as pl
from jax.experimental.pallas import tpu as pltpu
from jax.experimental.pallas import tpu_sc as plsc

BATCH = 4096
SEQ = 200
EMBED = 64

_NC, _NS = 2, 16
_NW = _NC * _NS
_BROWS_W = BATCH // _NW
_BR = 1
_NG = _BROWS_W // _BR
_NSLOT = 4
_SPLITS = ((0, 104), (104, 96))


def _sc_embed(x, token_table, pos_table):
    mesh = plsc.VectorSubcoreMesh(
        core_axis_name="c", subcore_axis_name="s", num_cores=_NC, num_subcores=_NS
    )

    @functools.partial(
        pl.kernel,
        mesh=mesh,
        out_type=jax.ShapeDtypeStruct((BATCH, SEQ, EMBED), jnp.float32),
        scratch_types=[
            pltpu.VMEM((_BROWS_W, SEQ), jnp.int32),
            pltpu.VMEM_SHARED((SEQ + 8, 2 * EMBED), jnp.float32),
        ]
        + [pltpu.VMEM((_BR, SEQ, 2 * EMBED), jnp.float32)] * _NSLOT
        + [pltpu.SemaphoreType.DMA] * (3 * _NSLOT),
        compiler_params=pltpu.CompilerParams(use_tc_tiling_on_sc=False),
    )
    def k(x_hbm, tok_hbm, pos_hbm, out_hbm, idx_v, spos, *rest):
        bufs = rest[:_NSLOT]
        sems_i = rest[_NSLOT : 2 * _NSLOT]
        sems_t = rest[2 * _NSLOT : 3 * _NSLOT]
        sems_o = rest[3 * _NSLOT : 4 * _NSLOT]
        sid = lax.axis_index("s")
        wid = sid * _NC + lax.axis_index("c")
        row0 = wid * _BROWS_W
        pltpu.sync_copy(x_hbm.at[pl.ds(row0, _BROWS_W), :], idx_v)

        @pl.when(sid == 0)
        def _():
            pltpu.sync_copy(pos_hbm.at[pl.ds(0, SEQ + 8), :], spos)

        plsc.subcore_barrier()

        def fire_init(b):
            return [
                pltpu.async_copy(
                    spos.at[pl.ds(1, SEQ)], bufs[b].at[br], sems_i[b]
                )
                for br in range(_BR)
            ]

        def fire_tok(g, b):
            return [
                pltpu.async_copy(
                    tok_hbm.at[idx_v.at[g * _BR + br].at[pl.ds(off, ln)]],
                    bufs[b].at[br].at[pl.ds(off, ln)],
                    sems_t[b],
                    add=True,
                )
                for br in range(_BR)
                for off, ln in _SPLITS
            ]

        def fire_out(g, b):
            pltpu.async_copy(
                bufs[b].at[:, :, pl.ds(0, EMBED)],
                out_hbm.at[pl.ds(row0 + g * _BR, _BR)],
                sems_o[b],
            )

        def drain_out(b):
            pltpu.make_async_copy(
                bufs[b].at[:, :, pl.ds(0, EMBED)],
                out_hbm.at[pl.ds(row0, _BR)],
                sems_o[b],
            ).wait()

        def round_body(i, carry):
            g0 = i * _NSLOT
            d_i = [None] * _NSLOT
            d_t = [None] * _NSLOT
            for b in range(_NSLOT):

                @pl.when(i > 0)
                def _(b=b):
                    drain_out(b)

                d_i[b] = fire_init(b)
            for b in range(_NSLOT):
                for d in d_i[b]:
                    d.wait()
                d_t[b] = fire_tok(g0 + b, b)
            for b in range(_NSLOT):
                for d in d_t[b]:
                    d.wait()
                fire_out(g0 + b, b)
            return carry

        lax.fori_loop(0, _NG // _NSLOT, round_body, 0)
        for b in range(_NSLOT):
            drain_out(b)

    tok_pad = jnp.pad(token_table, ((0, 0), (0, EMBED)))
    pos_pad = jnp.pad(pos_table, ((0, 0), (0, EMBED)))
    return k(x, tok_pad, pos_pad)


def _mask_body(x_ref, m_ref):
    m_ref[...] = x_ref[...] != 0


def _mask(x):
    return pl.pallas_call(
        _mask_body,
        out_shape=jax.ShapeDtypeStruct((BATCH, SEQ), jnp.bool_),
        grid=(8,),
        in_specs=[pl.BlockSpec((BATCH // 8, SEQ), lambda i: (i, 0))],
        out_specs=pl.BlockSpec((BATCH // 8, SEQ), lambda i: (i, 0)),
    )(x)


def kernel(x, token_table, pos_table):
    out = _sc_embed(x, token_table, pos_table)
    mask = _mask(x)
    return out, mask


# speedup vs baseline: 1.2608x; 1.2608x over previous
import functools

import jax
import jax.numpy as jnp
from jax import lax
from jax.experimental import pallas as pl
from jax.experimental.pallas import tpu as pltpu
from jax.experimental.pallas import tpu_sc as plsc

BATCH = 4096
SEQ = 200
EMBED = 64

_NC, _NS = 2, 16
_NW = _NC * _NS
_BROWS_W = BATCH // _NW
_BR = 1
_NG = _BROWS_W // _BR
_NSLOT = 8
_SPLITS = ((0, 104), (104, 96))


def _sc_embed(x, token_table, pos_table):
    mesh = plsc.VectorSubcoreMesh(
        core_axis_name="c", subcore_axis_name="s", num_cores=_NC, num_subcores=_NS
    )

    @functools.partial(
        pl.kernel,
        mesh=mesh,
        out_type=jax.ShapeDtypeStruct((BATCH, SEQ, EMBED), jnp.float32),
        scratch_types=[
            pltpu.VMEM((_BROWS_W, SEQ), jnp.int32),
            pltpu.VMEM_SHARED((SEQ + 8, EMBED), jnp.float32),
        ]
        + [pltpu.VMEM((_BR, SEQ, EMBED), jnp.float32)] * _NSLOT
        + [pltpu.SemaphoreType.DMA] * (3 * _NSLOT),
        compiler_params=pltpu.CompilerParams(use_tc_tiling_on_sc=False),
    )
    def k(x_hbm, tok_hbm, pos_hbm, out_hbm, idx_v, spos, *rest):
        bufs = rest[:_NSLOT]
        sems_i = rest[_NSLOT : 2 * _NSLOT]
        sems_t = rest[2 * _NSLOT : 3 * _NSLOT]
        sems_o = rest[3 * _NSLOT : 4 * _NSLOT]
        sid = lax.axis_index("s")
        wid = sid * _NC + lax.axis_index("c")
        row0 = wid * _BROWS_W
        pltpu.sync_copy(x_hbm.at[pl.ds(row0, _BROWS_W), :], idx_v)

        @pl.when(sid == 0)
        def _():
            pltpu.sync_copy(pos_hbm.at[pl.ds(0, SEQ + 8), :], spos)

        plsc.subcore_barrier()

        def fire_init(b):
            return [
                pltpu.async_copy(
                    spos.at[pl.ds(1, SEQ)], bufs[b].at[br], sems_i[b]
                )
                for br in range(_BR)
            ]

        def fire_tok(g, b):
            return [
                pltpu.async_copy(
                    tok_hbm.at[idx_v.at[g * _BR + br].at[pl.ds(off, ln)]],
                    bufs[b].at[br].at[pl.ds(off, ln)],
                    sems_t[b],
                    add=True,
                )
                for br in range(_BR)
                for off, ln in _SPLITS
            ]

        def fire_out(g, b):
            pltpu.async_copy(
                bufs[b], out_hbm.at[pl.ds(row0 + g * _BR, _BR)], sems_o[b]
            )

        def drain_out(b):
            pltpu.make_async_copy(
                bufs[b], out_hbm.at[pl.ds(row0, _BR)], sems_o[b]
            ).wait()

        def round_body(i, carry):
            g0 = i * _NSLOT
            d_i = [None] * _NSLOT
            d_t = [None] * _NSLOT
            for b in range(_NSLOT):

                @pl.when(i > 0)
                def _(b=b):
                    drain_out(b)

                d_i[b] = fire_init(b)
            for b in range(_NSLOT):
                for d in d_i[b]:
                    d.wait()
                d_t[b] = fire_tok(g0 + b, b)
            for b in range(_NSLOT):
                for d in d_t[b]:
                    d.wait()
                fire_out(g0 + b, b)
            return carry

        lax.fori_loop(0, _NG // _NSLOT, round_body, 0)
        for b in range(_NSLOT):
            drain_out(b)

    return k(x, token_table, pos_table)


def _mask_body(x_ref, m_ref):
    m_ref[...] = x_ref[...] != 0


def _mask(x):
    return pl.pallas_call(
        _mask_body,
        out_shape=jax.ShapeDtypeStruct((BATCH, SEQ), jnp.bool_),
        grid=(8,),
        in_specs=[pl.BlockSpec((BATCH // 8, SEQ), lambda i: (i, 0))],
        out_specs=pl.BlockSpec((BATCH // 8, SEQ), lambda i: (i, 0)),
    )(x)


def kernel(x, token_table, pos_table):
    out = _sc_embed(x, token_table, pos_table)
    mask = _mask(x)
    return out, mask
